# initial kernel scaffold (unmeasured)
import jax
import jax.numpy as jnp
from jax import lax
from jax.experimental import pallas as pl
from jax.experimental.pallas import tpu as pltpu


def kernel(
    x,
):
    def body(*refs):
        pass

    out_shape = jax.ShapeDtypeStruct(..., jnp.float32)
    return pl.pallas_call(body, out_shape=out_shape)(...)


# baseline (device time: 76605 ns/iter reference)
import jax
import jax.numpy as jnp
from jax import lax
from jax.experimental import pallas as pl
from jax.experimental.pallas import tpu as pltpu

N_DEV = 16


def kernel(x):
    m_rows, n_cols = x.shape

    def body(x_ref, out_ref, comm_ref, send_sems, recv_sems):
        my = lax.axis_index("i")
        left = lax.rem(my + N_DEV - 1, N_DEV)
        right = lax.rem(my + 1, N_DEV)

        barrier_sem = pltpu.get_barrier_semaphore()
        for nbr in [left, right]:
            pl.semaphore_signal(
                barrier_sem, inc=1,
                device_id=(nbr,), device_id_type=pl.DeviceIdType.MESH,
            )
        pl.semaphore_wait(barrier_sem, 2)

        xv = x_ref[...].astype(jnp.float32)
        m_loc = jnp.max(xv, axis=1, keepdims=True)
        s_loc = jnp.sum(jnp.exp(xv - m_loc), axis=1, keepdims=True)
        comm_ref[my] = jnp.concatenate([m_loc, s_loc], axis=1)

        for h in range(N_DEV - 1):
            src_idx = lax.rem(my - h + N_DEV, N_DEV)
            in_idx = lax.rem(my - h - 1 + N_DEV, N_DEV)
            send = pltpu.make_async_remote_copy(
                src_ref=comm_ref.at[src_idx],
                dst_ref=comm_ref.at[src_idx],
                send_sem=send_sems.at[h],
                recv_sem=recv_sems.at[h],
                device_id=(right,),
                device_id_type=pl.DeviceIdType.MESH,
            )
            send.start()
            recv = pltpu.make_async_remote_copy(
                src_ref=comm_ref.at[in_idx],
                dst_ref=comm_ref.at[in_idx],
                send_sem=send_sems.at[h],
                recv_sem=recv_sems.at[h],
                device_id=(left,),
                device_id_type=pl.DeviceIdType.MESH,
            )
            recv.wait_recv()
            send.wait_send()

        m_all = comm_ref[:, :, 0:1]
        s_all = comm_ref[:, :, 1:2]
        m_glob = jnp.max(m_all, axis=0)
        s_glob = jnp.sum(s_all * jnp.exp(m_all - m_glob[None]), axis=0)
        out_ref[...] = (jnp.exp(xv - m_glob) / s_glob).astype(out_ref.dtype)

    return pl.pallas_call(
        body,
        out_shape=jax.ShapeDtypeStruct((m_rows, n_cols), jnp.float32),
        in_specs=[pl.BlockSpec(memory_space=pltpu.VMEM)],
        out_specs=pl.BlockSpec(memory_space=pltpu.VMEM),
        scratch_shapes=[
            pltpu.VMEM((N_DEV, m_rows, 2), jnp.float32),
            pltpu.SemaphoreType.DMA((N_DEV - 1,)),
            pltpu.SemaphoreType.DMA((N_DEV - 1,)),
        ],
        compiler_params=pltpu.CompilerParams(collective_id=0),
    )(x)


# device time: 10426 ns/iter; 7.3475x vs baseline; 7.3475x over previous
import jax
import jax.numpy as jnp
from jax import lax
from jax.experimental import pallas as pl
from jax.experimental.pallas import tpu as pltpu

N_DEV = 16


def kernel(x):
    m_rows, n_cols = x.shape

    def body(x_ref, out_ref, gather_ref, send_sems, recv_sems):
        my = lax.axis_index("i")

        barrier_sem = pltpu.get_barrier_semaphore()
        for off in range(1, N_DEV):
            peer = lax.rem(my + off, N_DEV)
            pl.semaphore_signal(
                barrier_sem, inc=1,
                device_id=(peer,), device_id_type=pl.DeviceIdType.MESH,
            )
        pl.semaphore_wait(barrier_sem, N_DEV - 1)

        xv = x_ref[...].astype(jnp.float32)
        m_loc = jnp.max(xv, axis=1, keepdims=True)
        s_loc = jnp.sum(jnp.exp(xv - m_loc), axis=1, keepdims=True)
        pad = jnp.zeros((m_rows, 6), jnp.float32)
        stats_row = jnp.transpose(
            jnp.concatenate([m_loc, s_loc, pad], axis=1), (1, 0)
        )
        gather_ref[my] = stats_row

        sends = []
        for off in range(1, N_DEV):
            peer = lax.rem(my + off, N_DEV)
            rdma = pltpu.make_async_remote_copy(
                src_ref=gather_ref.at[my],
                dst_ref=gather_ref.at[my],
                send_sem=send_sems.at[peer],
                recv_sem=recv_sems.at[my],
                device_id=(peer,),
                device_id_type=pl.DeviceIdType.MESH,
            )
            rdma.start()
            sends.append(rdma)
        for off in range(1, N_DEV):
            peer = lax.rem(my + off, N_DEV)
            recv = pltpu.make_async_remote_copy(
                src_ref=gather_ref.at[peer],
                dst_ref=gather_ref.at[peer],
                send_sem=send_sems.at[peer],
                recv_sem=recv_sems.at[peer],
                device_id=(peer,),
                device_id_type=pl.DeviceIdType.MESH,
            )
            recv.wait_recv()
        for rdma in sends:
            rdma.wait_send()

        g = gather_ref[...]
        m_all = g[:, 0:1, :]
        s_all = g[:, 1:2, :]
        m_glob = jnp.max(m_all, axis=0)
        s_glob = jnp.sum(s_all * jnp.exp(m_all - m_glob[None]), axis=0)
        pad_row = jnp.zeros((6, m_rows), jnp.float32)
        res_col = jnp.transpose(
            jnp.concatenate([m_glob, s_glob, pad_row], axis=0), (1, 0)
        )
        m_col = res_col[:, 0:1]
        s_col = res_col[:, 1:2]
        out_ref[...] = (jnp.exp(xv - m_col) / s_col).astype(out_ref.dtype)

    return pl.pallas_call(
        body,
        out_shape=jax.ShapeDtypeStruct((m_rows, n_cols), jnp.float32),
        in_specs=[pl.BlockSpec(memory_space=pltpu.VMEM)],
        out_specs=pl.BlockSpec(memory_space=pltpu.VMEM),
        scratch_shapes=[
            pltpu.VMEM((N_DEV, 8, m_rows), jnp.float32),
            pltpu.SemaphoreType.DMA((N_DEV,)),
            pltpu.SemaphoreType.DMA((N_DEV,)),
        ],
        compiler_params=pltpu.CompilerParams(collective_id=0),
    )(x)
